# baseline (speedup 1.0000x reference)
import functools

import jax
import jax.numpy as jnp
from jax import lax
from jax.experimental import pallas as pl
from jax.experimental.pallas import tpu as pltpu
from jax.experimental.pallas import tpu_sc as plsc

E = 128
NC = 2
NS = 16
NW = NC * NS
CH = 320
GRP = CH // 16


def _rsqrt_newton(x):
    i = plsc.bitcast(x, jnp.int32)
    i = jnp.int32(0x5F3759DF) - lax.shift_right_logical(i, 1)
    y = plsc.bitcast(i, jnp.float32)
    half = x * 0.5
    for _ in range(3):
        y = y * (1.5 - half * y * y)
    return y


def _fused_body(l_seq, seq_hbm, table_hbm, pe_hbm, gamma_hbm, beta_hbm,
                out_hbm, idx_v, rows_a, rows_b, pe_v, gb_v, g16_v, b16_v,
                scr_v, sem_ga, sem_gb, sem_oa, sem_ob):
    t_total = seq_hbm.shape[0]
    tpw = t_total // NW
    nch = tpw // CH
    wid = lax.axis_index("s") * NC + lax.axis_index("c")
    base = wid * tpw

    pltpu.sync_copy(seq_hbm.at[pl.ds(base, tpw)], idx_v)
    pltpu.sync_copy(pe_hbm, pe_v)
    pltpu.sync_copy(gamma_hbm, gb_v.at[0])
    pltpu.sync_copy(beta_hbm, gb_v.at[1])
    zero16 = jnp.zeros((16,), jnp.int32)
    one16 = jnp.full((16,), 1, jnp.int32)
    for f in range(E):
        colf = jnp.full((16,), f, jnp.int32)
        g16_v[pl.ds(f * 16, 16)] = plsc.load_gather(gb_v, [zero16, colf])
        b16_v[pl.ds(f * 16, 16)] = plsc.load_gather(gb_v, [one16, colf])

    bufs = (rows_a, rows_b)
    g_sems = (sem_ga, sem_gb)
    o_sems = (sem_oa, sem_ob)

    def start_gather(c, b):
        pltpu.async_copy(table_hbm.at[idx_v.at[pl.ds(c * CH, CH)]],
                         bufs[b], g_sems[b])

    def wait_gather(c, b):
        pltpu.make_async_copy(table_hbm.at[idx_v.at[pl.ds(c * CH, CH)]],
                              bufs[b], g_sems[b]).wait()

    def start_out(c, b):
        pltpu.async_copy(bufs[b], out_hbm.at[pl.ds(base + c * CH, CH)],
                         o_sems[b])

    def wait_out(c, b):
        pltpu.make_async_copy(bufs[b], out_hbm.at[pl.ds(base + c * CH, CH)],
                              o_sems[b]).wait()

    iota = lax.iota(jnp.int32, 16)

    def compute_chunk(c, b):
        rows = bufs[b]

        def group_body(g, _):
            rowv = g * 16 + iota
            tid = idx_v[pl.ds(c * CH + g * 16, 16)]
            mz = jnp.where(tid == 0, 0.0, 1.0)
            pos = lax.rem(base + c * CH + rowv, l_seq)
            s = jnp.zeros((16,), jnp.float32)
            q = jnp.zeros((16,), jnp.float32)
            for f in range(E):
                colf = jnp.full((16,), f, jnp.int32)
                a = plsc.load_gather(rows, [rowv, colf])
                p = plsc.load_gather(pe_v, [pos, colf])
                e = a * mz + p
                s = s + e
                q = q + e * e
                scr_v[pl.ds(f * 16, 16)] = e
            mean = s * (1.0 / E)
            var = q * (1.0 / E) - mean * mean
            rstd = _rsqrt_newton(var + 1e-12)
            for f in range(E):
                colf = jnp.full((16,), f, jnp.int32)
                e = scr_v[pl.ds(f * 16, 16)]
                o = (e - mean) * rstd * g16_v[pl.ds(f * 16, 16)] + b16_v[pl.ds(f * 16, 16)]
                plsc.store_scatter(rows, [rowv, colf], o)
            return 0

        lax.fori_loop(0, GRP, group_body, 0)

    start_gather(0, 0)

    def chunk_pair(i, _):
        c2 = i * 2
        for b in (0, 1):
            c = c2 + b
            wait_gather(c, b)
            nb = 1 - b

            @pl.when(c >= 1)
            def _():
                wait_out(c - 1, nb)

            @pl.when(c + 1 < nch)
            def _():
                start_gather(c + 1, nb)

            compute_chunk(c, b)
            start_out(c, b)
        return 0

    lax.fori_loop(0, nch // 2, chunk_pair, 0)
    wait_out(nch - 1, (nch - 1) % 2)


def kernel(sequence, table, gamma, beta, pe):
    b, l = sequence.shape
    t_total = b * l
    seq_flat = sequence.reshape(-1).astype(jnp.int32)
    pe_l = pe[:l]
    mesh = plsc.VectorSubcoreMesh(core_axis_name="c", subcore_axis_name="s")
    fn = pl.kernel(
        functools.partial(_fused_body, l),
        out_type=jax.ShapeDtypeStruct((t_total, E), jnp.float32),
        mesh=mesh,
        compiler_params=pltpu.CompilerParams(needs_layout_passes=False),
        scratch_types=[
            pltpu.VMEM((t_total // NW,), jnp.int32),
            pltpu.VMEM((CH, E), jnp.float32),
            pltpu.VMEM((CH, E), jnp.float32),
            pltpu.VMEM((l, E), jnp.float32),
            pltpu.VMEM((2, E), jnp.float32),
            pltpu.VMEM((E * 16,), jnp.float32),
            pltpu.VMEM((E * 16,), jnp.float32),
            pltpu.VMEM((E * 16,), jnp.float32),
            pltpu.SemaphoreType.DMA,
            pltpu.SemaphoreType.DMA,
            pltpu.SemaphoreType.DMA,
            pltpu.SemaphoreType.DMA,
        ],
    )
    out = fn(seq_flat, table, pe_l, gamma, beta)
    return out.reshape(b, l, E)


# speedup vs baseline: 5.7364x; 5.7364x over previous
import functools

import jax
import jax.numpy as jnp
from jax import lax
from jax.experimental import pallas as pl
from jax.experimental.pallas import tpu as pltpu
from jax.experimental.pallas import tpu_sc as plsc

E = 128
EJ = E // 16
NC = 2
NS = 16
NW = NC * NS
CH = 320
GRP = CH // 16


def _rsqrt_newton(x):
    i = plsc.bitcast(x, jnp.int32)
    i = jnp.int32(0x5F3759DF) - lax.shift_right_logical(i, 1)
    y = plsc.bitcast(i, jnp.float32)
    half = x * 0.5
    for _ in range(3):
        y = y * (1.5 - half * y * y)
    return y


def _fused_body(l_seq, seq_hbm, table_hbm, pe_hbm, gamma_hbm, beta_hbm,
                out_hbm, idx_v, rows_a, rows_b, pe_v, gb_v,
                sem_ga, sem_gb, sem_oa, sem_ob):
    t_total = seq_hbm.shape[0]
    tpw = t_total // NW
    nch = tpw // CH
    wid = lax.axis_index("s") * NC + lax.axis_index("c")
    base = wid * tpw

    pltpu.sync_copy(seq_hbm.at[pl.ds(base, tpw)], idx_v)
    pltpu.sync_copy(pe_hbm, pe_v)
    pltpu.sync_copy(gamma_hbm, gb_v.at[0])
    pltpu.sync_copy(beta_hbm, gb_v.at[1])

    bufs = (rows_a, rows_b)
    g_sems = (sem_ga, sem_gb)
    o_sems = (sem_oa, sem_ob)

    def start_gather(c, b):
        pltpu.async_copy(table_hbm.at[idx_v.at[pl.ds(c * CH, CH)]],
                         bufs[b], g_sems[b])

    def wait_gather(c, b):
        pltpu.make_async_copy(table_hbm.at[idx_v.at[pl.ds(c * CH, CH)]],
                              bufs[b], g_sems[b]).wait()

    def start_out(c, b):
        pltpu.async_copy(bufs[b], out_hbm.at[pl.ds(base + c * CH, CH)],
                         o_sems[b])

    def wait_out(c, b):
        pltpu.make_async_copy(bufs[b], out_hbm.at[pl.ds(base + c * CH, CH)],
                              o_sems[b]).wait()

    def compute_chunk(c, b):
        rows = bufs[b]
        gs = [gb_v[0, pl.ds(j * 16, 16)] for j in range(EJ)]
        bs = [gb_v[1, pl.ds(j * 16, 16)] for j in range(EJ)]

        def group_body(g, _):
            tid16 = idx_v[pl.ds(c * CH + g * 16, 16)]
            pos0 = lax.rem(base + c * CH + g * 16, l_seq)
            for k in range(16):
                t = g * 16 + k
                tid = jnp.full((16,), tid16[k], jnp.int32)
                mz = jnp.where(tid == 0, 0.0, 1.0)
                pos = pos0 + k
                pos = jnp.where(pos >= l_seq, pos - l_seq, pos)
                e = [rows[t, pl.ds(j * 16, 16)] * mz + pe_v[pos, pl.ds(j * 16, 16)]
                     for j in range(EJ)]
                s = e[0]
                q = e[0] * e[0]
                for j in range(1, EJ):
                    s = s + e[j]
                    q = q + e[j] * e[j]
                st = jnp.full((16,), jnp.sum(s), jnp.float32)
                qt = jnp.full((16,), jnp.sum(q), jnp.float32)
                mean = st * (1.0 / E)
                var = qt * (1.0 / E) - mean * mean
                rstd = _rsqrt_newton(var + 1e-12)
                for j in range(EJ):
                    rows[t, pl.ds(j * 16, 16)] = (e[j] - mean) * rstd * gs[j] + bs[j]
            return 0

        lax.fori_loop(0, GRP, group_body, 0)

    start_gather(0, 0)

    def chunk_pair(i, _):
        c2 = i * 2
        for b in (0, 1):
            c = c2 + b
            wait_gather(c, b)
            nb = 1 - b

            @pl.when(c >= 1)
            def _():
                wait_out(c - 1, nb)

            @pl.when(c + 1 < nch)
            def _():
                start_gather(c + 1, nb)

            compute_chunk(c, b)
            start_out(c, b)
        return 0

    lax.fori_loop(0, nch // 2, chunk_pair, 0)
    wait_out(nch - 1, (nch - 1) % 2)


def kernel(sequence, table, gamma, beta, pe):
    b, l = sequence.shape
    t_total = b * l
    seq_flat = sequence.reshape(-1).astype(jnp.int32)
    pe_l = pe[:l]
    mesh = plsc.VectorSubcoreMesh(core_axis_name="c", subcore_axis_name="s")
    fn = pl.kernel(
        functools.partial(_fused_body, l),
        out_type=jax.ShapeDtypeStruct((t_total, E), jnp.float32),
        mesh=mesh,
        compiler_params=pltpu.CompilerParams(needs_layout_passes=False),
        scratch_types=[
            pltpu.VMEM((t_total // NW,), jnp.int32),
            pltpu.VMEM((CH, E), jnp.float32),
            pltpu.VMEM((CH, E), jnp.float32),
            pltpu.VMEM((l, E), jnp.float32),
            pltpu.VMEM((2, E), jnp.float32),
            pltpu.SemaphoreType.DMA,
            pltpu.SemaphoreType.DMA,
            pltpu.SemaphoreType.DMA,
            pltpu.SemaphoreType.DMA,
        ],
    )
    out = fn(seq_flat, table, pe_l, gamma, beta)
    return out.reshape(b, l, E)


# speedup vs baseline: 11.4010x; 1.9875x over previous
import functools

import jax
import jax.numpy as jnp
from jax import lax
from jax.experimental import pallas as pl
from jax.experimental.pallas import tpu as pltpu
from jax.experimental.pallas import tpu_sc as plsc

E = 128
EJ = E // 16
NC = 2
NS = 16
NW = NC * NS
CH = 320
GRP = CH // 16


def _rsqrt_newton(x):
    i = plsc.bitcast(x, jnp.int32)
    i = jnp.int32(0x5F3759DF) - lax.shift_right_logical(i, 1)
    y = plsc.bitcast(i, jnp.float32)
    half = x * 0.5
    for _ in range(3):
        y = y * (1.5 - half * y * y)
    return y


def _fused_body(l_seq, seq_hbm, table_hbm, pe_hbm, gamma_hbm, beta_hbm,
                out_hbm, idx_v, rows_a, rows_b, pe_v, gb_v,
                sem_ga, sem_gb, sem_oa, sem_ob):
    t_total = seq_hbm.shape[0]
    tpw = t_total // NW
    nch = tpw // CH
    wid = lax.axis_index("s") * NC + lax.axis_index("c")
    base = wid * tpw

    pltpu.sync_copy(seq_hbm.at[pl.ds(base, tpw)], idx_v)
    pltpu.sync_copy(pe_hbm, pe_v)
    pltpu.sync_copy(gamma_hbm, gb_v.at[0])
    pltpu.sync_copy(beta_hbm, gb_v.at[1])

    bufs = (rows_a, rows_b)
    g_sems = (sem_ga, sem_gb)
    o_sems = (sem_oa, sem_ob)

    def start_gather(c, b):
        pltpu.async_copy(table_hbm.at[idx_v.at[pl.ds(c * CH, CH)]],
                         bufs[b], g_sems[b])

    def wait_gather(c, b):
        pltpu.make_async_copy(table_hbm.at[idx_v.at[pl.ds(c * CH, CH)]],
                              bufs[b], g_sems[b]).wait()

    def start_out(c, b):
        pltpu.async_copy(bufs[b], out_hbm.at[pl.ds(base + c * CH, CH)],
                         o_sems[b])

    def wait_out(c, b):
        pltpu.make_async_copy(bufs[b], out_hbm.at[pl.ds(base + c * CH, CH)],
                              o_sems[b]).wait()

    def compute_chunk(c, b):
        rows = bufs[b]
        gs = [gb_v[0, pl.ds(j * 16, 16)] for j in range(EJ)]
        bs = [gb_v[1, pl.ds(j * 16, 16)] for j in range(EJ)]

        lane = lax.iota(jnp.int32, 16)

        def group_body(g, _):
            tid16 = idx_v[pl.ds(c * CH + g * 16, 16)]
            mz16 = jnp.where(tid16 == 0, 0.0, 1.0)
            pos0 = lax.rem(base + c * CH + g * 16, l_seq)
            sums = jnp.zeros((16,), jnp.float32)
            sqs = jnp.zeros((16,), jnp.float32)
            for k in range(16):
                t = g * 16 + k
                mzk = jnp.full((16,), mz16[k], jnp.float32)
                pos = pos0 + k
                pos = jnp.where(pos >= l_seq, pos - l_seq, pos)
                e = [rows[t, pl.ds(j * 16, 16)] * mzk + pe_v[pos, pl.ds(j * 16, 16)]
                     for j in range(EJ)]
                s = e[0]
                q = e[0] * e[0]
                for j in range(1, EJ):
                    s = s + e[j]
                    q = q + e[j] * e[j]
                for j in range(EJ):
                    rows[t, pl.ds(j * 16, 16)] = e[j]
                sk = jnp.full((16,), jnp.sum(s), jnp.float32)
                qk = jnp.full((16,), jnp.sum(q), jnp.float32)
                sums = jnp.where(lane == k, sk, sums)
                sqs = jnp.where(lane == k, qk, sqs)
            mean16 = sums * (1.0 / E)
            var16 = sqs * (1.0 / E) - mean16 * mean16
            rstd16 = _rsqrt_newton(var16 + 1e-12)
            for k in range(16):
                t = g * 16 + k
                mk = jnp.full((16,), mean16[k], jnp.float32)
                rk = jnp.full((16,), rstd16[k], jnp.float32)
                for j in range(EJ):
                    ej = rows[t, pl.ds(j * 16, 16)]
                    rows[t, pl.ds(j * 16, 16)] = (ej - mk) * rk * gs[j] + bs[j]
            return 0

        lax.fori_loop(0, GRP, group_body, 0)

    start_gather(0, 0)

    def chunk_pair(i, _):
        c2 = i * 2
        for b in (0, 1):
            c = c2 + b
            wait_gather(c, b)
            nb = 1 - b

            @pl.when(c >= 1)
            def _():
                wait_out(c - 1, nb)

            @pl.when(c + 1 < nch)
            def _():
                start_gather(c + 1, nb)

            compute_chunk(c, b)
            start_out(c, b)
        return 0

    lax.fori_loop(0, nch // 2, chunk_pair, 0)
    wait_out(nch - 1, (nch - 1) % 2)


def kernel(sequence, table, gamma, beta, pe):
    b, l = sequence.shape
    t_total = b * l
    seq_flat = sequence.reshape(-1).astype(jnp.int32)
    pe_l = pe[:l]
    mesh = plsc.VectorSubcoreMesh(core_axis_name="c", subcore_axis_name="s")
    fn = pl.kernel(
        functools.partial(_fused_body, l),
        out_type=jax.ShapeDtypeStruct((t_total, E), jnp.float32),
        mesh=mesh,
        compiler_params=pltpu.CompilerParams(needs_layout_passes=False),
        scratch_types=[
            pltpu.VMEM((t_total // NW,), jnp.int32),
            pltpu.VMEM((CH, E), jnp.float32),
            pltpu.VMEM((CH, E), jnp.float32),
            pltpu.VMEM((l, E), jnp.float32),
            pltpu.VMEM((2, E), jnp.float32),
            pltpu.SemaphoreType.DMA,
            pltpu.SemaphoreType.DMA,
            pltpu.SemaphoreType.DMA,
            pltpu.SemaphoreType.DMA,
        ],
    )
    out = fn(seq_flat, table, pe_l, gamma, beta)
    return out.reshape(b, l, E)
